# baseline (speedup 1.0000x reference)
import functools

import jax
import jax.numpy as jnp
from jax import lax
from jax.experimental import pallas as pl
from jax.experimental.pallas import tpu as pltpu
from jax.experimental.pallas import tpu_sc as plsc

L_PLANES = 16
B = 8
T_B = 512
A = 512
T_BLK = 128
N_WORKERS = 32
WINDOW = 15
SENTINEL = 1024


def _sc_body_half(half, ti_hbm, vl_hbm, aidx_hbm, sumt_hbm, cam_v, vl_v, out_v,
                  sumt_v):
    wid = lax.axis_index("s") * 2 + lax.axis_index("c")
    b = half * (B // 2) + wid // 8
    t0 = (wid % 8) * (T_BLK // 2)

    pltpu.sync_copy(ti_hbm.at[b], cam_v)
    pltpu.sync_copy(vl_hbm, vl_v)

    lane = lax.iota(jnp.int32, 16)
    t_vec = plsc.load_gather(vl_v, [jnp.full((16,), b, jnp.int32)])

    tmax = 2 * t_vec - 16
    lane_ok = lane < WINDOW

    def row(g, _):
        for u in range(4):
            r = g * 4 + u
            t = t0 + r
            start = jnp.minimum(tmax, jnp.maximum(0, 2 * t - 7))
            idx = start + lane
            idx = jnp.where(idx < 0, idx + 1024, idx)
            a = plsc.load_gather(cam_v, [idx])
            valid = (t < t_vec) & lane_ok
            a = jnp.where(valid, a, SENTINEL)
            out_v[pl.ds(r * 16, 16)] = a
        return _

    lax.fori_loop(0, T_BLK // 8, row, None)
    pltpu.sync_copy(out_v, aidx_hbm.at[b - half * (B // 2),
                                       pl.ds(t0 * 16, (T_BLK // 2) * 16)])

    @pl.when((wid == 0) & (half == 0))
    def _():
        idx8 = jnp.where(lane < B, lane, 0)
        vals = plsc.load_gather(vl_v, [idx8])
        vals = jnp.where(lane < B, vals, 0)
        s = jnp.sum(vals)
        sumt_v[...] = jnp.full((16,), s, jnp.int32)
        pltpu.sync_copy(sumt_v, sumt_hbm)


def _sc_gather_half(half, ti, vl):
    fn = functools.partial(
        pl.kernel,
        mesh=plsc.VectorSubcoreMesh(
            core_axis_name="c", subcore_axis_name="s", num_cores=2
        ),
        compiler_params=pltpu.CompilerParams(needs_layout_passes=False),
        out_type=(
            jax.ShapeDtypeStruct((B // 2, T_B * 16), jnp.int32),
            jax.ShapeDtypeStruct((16,), jnp.int32),
        ),
        scratch_types=[
            pltpu.VMEM((1024,), jnp.int32),
            pltpu.VMEM((8,), jnp.int32),
            pltpu.VMEM(((T_BLK // 2) * 16,), jnp.int32),
            pltpu.VMEM((16,), jnp.int32),
        ],
    )(functools.partial(_sc_body_half, half))
    return fn(ti, vl)


TC_BLK = 256


def _tc_body_half(half, aidx_ref, sumt_ref, acc_ref, att_ref, out_ref):
    b = pl.program_id(0)
    tb = pl.program_id(1)
    n_tb = T_B // TC_BLK

    @pl.when((b == 0) & (tb == 0))
    def _():
        out_ref[...] = acc_ref[...]

    s = att_ref[0, 0, 0]
    for lh in range(1, L_PLANES):
        s = s + att_ref[lh // 8, 0, lh % 8]
    logp = jnp.log(s * (1.0 / L_PLANES) + 1e-8)
    cols = lax.broadcasted_iota(jnp.int32, (TC_BLK, A), 1)
    counts = jnp.zeros((TC_BLK, A), jnp.float32)
    for j in range(WINDOW):
        aj = aidx_ref[0, :, j : j + 1]
        counts += jnp.where(aj == cols, 1.0, 0.0)
    out_ref[...] = out_ref[...] + jnp.sum(counts * logp)

    if half == 1:
        @pl.when((b == B // 2 - 1) & (tb == n_tb - 1))
        def _():
            out_ref[...] = -out_ref[...] / sumt_ref[0, 0].astype(jnp.float32)


def _tc_loss_half(half, aidx_h, sumt, att, acc):
    def body(aidx_ref, sumt_ref, acc_ref, att_ref, out_ref):
        _tc_body_half(half, aidx_ref, sumt_ref, acc_ref, att_ref, out_ref)

    return pl.pallas_call(
        body,
        grid=(B // 2, T_B // TC_BLK),
        in_specs=[
            pl.BlockSpec((1, TC_BLK, 16), lambda b, tb: (b, tb, 0)),
            pl.BlockSpec((1, 16), lambda b, tb: (0, 0)),
            pl.BlockSpec((1, 1), lambda b, tb: (0, 0)),
            pl.BlockSpec(
                (2, 1, 8, TC_BLK, A),
                lambda b, tb: (0, half * (B // 2) + b, 0, tb, 0),
            ),
        ],
        out_specs=pl.BlockSpec((1, 1), lambda b, tb: (0, 0)),
        out_shape=jax.ShapeDtypeStruct((1, 1), jnp.float32),
    )(aidx_h, sumt, acc, att)


def kernel(attention_scores, target_indices, video_length):
    ti = target_indices.astype(jnp.int32)
    vl = video_length.astype(jnp.int32)

    aidx0_flat, sumt = _sc_gather_half(0, ti, vl)
    aidx1_flat, _unused = _sc_gather_half(1, ti, vl)
    aidx0 = aidx0_flat.reshape(B // 2, T_B, 16)
    aidx1 = aidx1_flat.reshape(B // 2, T_B, 16)
    sumt2d = sumt.reshape(1, 16)

    zero = jnp.zeros((1, 1), jnp.float32)
    acc0 = _tc_loss_half(0, aidx0, sumt2d, attention_scores, zero)
    return _tc_loss_half(1, aidx1, sumt2d, attention_scores, acc0)[0, 0]


# speedup vs baseline: 1.1128x; 1.1128x over previous
import functools

import jax
import jax.numpy as jnp
from jax import lax
from jax.experimental import pallas as pl
from jax.experimental.pallas import tpu as pltpu
from jax.experimental.pallas import tpu_sc as plsc

L_PLANES = 16
B = 8
T_B = 512
A = 512
T_BLK = 128
N_WORKERS = 32
WINDOW = 15
SENTINEL = 1024


def _sc_body(ti_hbm, vl_hbm, aidx_hbm, sumt_hbm, cam_v, vl_v, out_v, sumt_v):
    wid = lax.axis_index("s") * 2 + lax.axis_index("c")
    b = wid // 4
    t0 = (wid % 4) * T_BLK

    pltpu.sync_copy(ti_hbm.at[b], cam_v)
    pltpu.sync_copy(vl_hbm, vl_v)

    lane = lax.iota(jnp.int32, 16)
    t_vec = plsc.load_gather(vl_v, [jnp.full((16,), b, jnp.int32)])

    tmax = 2 * t_vec - 16
    lane_ok = lane < WINDOW

    def row(g, _):
        for u in range(4):
            r = g * 4 + u
            t = t0 + r
            start = jnp.minimum(tmax, jnp.maximum(0, 2 * t - 7))
            idx = start + lane
            idx = jnp.where(idx < 0, idx + 1024, idx)
            a = plsc.load_gather(cam_v, [idx])
            valid = (t < t_vec) & lane_ok
            a = jnp.where(valid, a, SENTINEL)
            out_v[pl.ds(r * 16, 16)] = a
        return _

    lax.fori_loop(0, T_BLK // 4, row, None)
    pltpu.sync_copy(out_v, aidx_hbm.at[b, pl.ds(t0 * 16, T_BLK * 16)])

    @pl.when(wid == 0)
    def _():
        idx8 = jnp.where(lane < B, lane, 0)
        vals = plsc.load_gather(vl_v, [idx8])
        vals = jnp.where(lane < B, vals, 0)
        s = jnp.sum(vals)
        sumt_v[...] = jnp.full((16,), s, jnp.int32)
        pltpu.sync_copy(sumt_v, sumt_hbm)


def _sc_gather(ti, vl16):
    fn = functools.partial(
        pl.kernel,
        mesh=plsc.VectorSubcoreMesh(
            core_axis_name="c", subcore_axis_name="s", num_cores=2
        ),
        compiler_params=pltpu.CompilerParams(needs_layout_passes=False),
        out_type=(
            jax.ShapeDtypeStruct((B, T_B * 16), jnp.int32),
            jax.ShapeDtypeStruct((16,), jnp.int32),
        ),
        scratch_types=[
            pltpu.VMEM((1024,), jnp.int32),
            pltpu.VMEM((8,), jnp.int32),
            pltpu.VMEM((T_BLK * 16,), jnp.int32),
            pltpu.VMEM((16,), jnp.int32),
        ],
    )(_sc_body)
    return fn(ti, vl16)


TC_BLK = 256


def _tc_body(aidx_ref, sumt_ref, att_ref, out_ref):
    b = pl.program_id(0)
    tb = pl.program_id(1)
    n_tb = T_B // TC_BLK

    @pl.when((b == 0) & (tb == 0))
    def _():
        out_ref[...] = jnp.zeros_like(out_ref)

    s = att_ref[0, 0, 0]
    for lh in range(1, L_PLANES):
        s = s + att_ref[lh // 8, 0, lh % 8]
    logp = jnp.log(s * (1.0 / L_PLANES) + 1e-8)
    cols = lax.broadcasted_iota(jnp.int32, (TC_BLK, A), 1)
    counts = jnp.zeros((TC_BLK, A), jnp.float32)
    for j in range(WINDOW):
        aj = aidx_ref[0, :, j : j + 1]
        counts += jnp.where(aj == cols, 1.0, 0.0)
    out_ref[...] = out_ref[...] + jnp.sum(counts * logp)

    @pl.when((b == B - 1) & (tb == n_tb - 1))
    def _():
        out_ref[...] = -out_ref[...] / sumt_ref[0, 0].astype(jnp.float32)


def _tc_loss(aidx, sumt, att):
    return pl.pallas_call(
        _tc_body,
        grid=(B, T_B // TC_BLK),
        in_specs=[
            pl.BlockSpec((1, TC_BLK, 16), lambda b, tb: (b, tb, 0)),
            pl.BlockSpec((1, 16), lambda b, tb: (0, 0)),
            pl.BlockSpec(
                (2, 1, 8, TC_BLK, A),
                lambda b, tb: (0, b, 0, tb, 0),
            ),
        ],
        out_specs=pl.BlockSpec((1, 1), lambda b, tb: (0, 0)),
        out_shape=jax.ShapeDtypeStruct((1, 1), jnp.float32),
    )(aidx, sumt, att)


def kernel(attention_scores, target_indices, video_length):
    ti = target_indices.astype(jnp.int32)
    vl = video_length.astype(jnp.int32)

    aidx_flat, sumt = _sc_gather(ti, vl)
    aidx = aidx_flat.reshape(B, T_B, 16)

    return _tc_loss(aidx, sumt.reshape(1, 16), attention_scores)[0, 0]
